# Initial kernel scaffold; baseline (speedup 1.0000x reference)
import jax
import jax.numpy as jnp
from jax.experimental import pallas as pl


def kernel(x):
    raise NotImplementedError("write your pallas kernel here")


# speedup vs baseline: 1.4460x; 1.4460x over previous
import jax
import jax.numpy as jnp
from jax.experimental import pallas as pl
from jax.experimental.pallas import tpu as pltpu


def _argmin_body(x_ref, o_ref):
    xb = x_ref[0]
    N, D = xb.shape
    R = 8
    x3 = xb.reshape(N // R, R, D)
    mv = jnp.min(x3, axis=0)
    iota = jax.lax.broadcasted_iota(jnp.int32, (N // R, R, D), 0)
    cand = jnp.where(x3 == mv[None], iota, jnp.int32(N))
    mi = jnp.min(cand, axis=0)
    m = jnp.min(mv, axis=0)
    sub = jax.lax.broadcasted_iota(jnp.int32, (R, D), 0)
    full_idx = mi * R + sub
    idx_cand = jnp.where(mv == m[None], full_idx, jnp.int32(N))
    o_ref[0, 0, :] = jnp.min(idx_cand, axis=0)


def kernel(x):
    B, N, D = x.shape
    out = pl.pallas_call(
        _argmin_body,
        grid=(B,),
        in_specs=[pl.BlockSpec((1, N, D), lambda b: (b, 0, 0))],
        out_specs=pl.BlockSpec((1, 1, D), lambda b: (b, 0, 0)),
        out_shape=jax.ShapeDtypeStruct((B, 1, D), jnp.int32),
        compiler_params=pltpu.CompilerParams(
            dimension_semantics=("arbitrary",),
        ),
    )(x)
    return out.reshape(B, D).astype(jnp.int64)
